# baseline (speedup 1.0000x reference)
import functools

import jax
import jax.numpy as jnp
from jax import lax
from jax.experimental import pallas as pl
from jax.experimental.pallas import tpu as pltpu
from jax.experimental.pallas import tpu_sc as plsc

_D = 64
_K = 1024
_BLK = 512

_NC = 2
_NS = 16
_NW = _NC * _NS
_CH = 128


def _argmin_body(z_ref, cb_ref, x2_ref, w2_ref, idx_ref, *, blk_e):
    cb = cb_ref[...]
    ze = z_ref[...]
    x2_all = x2_ref[...]
    w2 = w2_ref[...]
    lane = jax.lax.broadcasted_iota(jnp.int32, (blk_e, _K), 1)

    cols = []
    for s in range(8):
        z = jax.lax.slice(ze, (0, s * _D), (blk_e, (s + 1) * _D))
        x2 = jax.lax.slice(x2_all, (0, s), (blk_e, s + 1))

        zc = jax.lax.dot_general(z, cb, (((1,), (1,)), ((), ())),
                                 preferred_element_type=jnp.float32)
        d2 = jnp.maximum(x2 - 2.0 * zc + w2, 0.0)
        dist = jnp.sqrt(d2)

        minval = jnp.min(dist, axis=1, keepdims=True)
        cols.append(jnp.min(jnp.where(dist == minval, lane, _K), axis=1,
                            keepdims=True))
    idx_ref[...] = jnp.concatenate(cols, axis=1)


def _sc_vq_body(idx_hbm, cb_hbm, z_hbm, ones_hbm, zeros_hbm,
                quant_hbm, hist_hbm, sse_hbm,
                idx_v, rows_v, z_v, out_v, ones_v, sse_v, hist_sh, sem):
    c_id = lax.axis_index("c")
    s_id = lax.axis_index("s")
    wid = s_id * _NC + c_id
    n_rows = z_hbm.shape[0] * (z_hbm.shape[1] // _D)
    rpw = n_rows // _NW
    nch = rpw // _CH
    base = wid * rpw

    pltpu.sync_copy(ones_hbm, ones_v)

    @pl.when(s_id == 0)
    def _():
        pltpu.sync_copy(zeros_hbm, hist_sh)
    plsc.subcore_barrier()

    ze_per_ch = _CH // 8

    def _chunk(c, acc):
        row0 = pl.multiple_of(base + c * _CH, _CH)
        ze0 = pl.multiple_of(row0 // 8, _CH // 8)
        pltpu.sync_copy(idx_hbm.at[pl.ds(row0, _CH)], idx_v)
        pltpu.async_copy(cb_hbm.at[idx_v], rows_v, sem).wait()
        pltpu.sync_copy(z_hbm.at[pl.ds(ze0, ze_per_ch), :], z_v)

        def _row(r, a):
            for s in range(8):
                for k in range(_D // 16):
                    col = s * _D + k * 16
                    zz = z_v[r, pl.ds(col, 16)]
                    qq = rows_v[r * 8 + s, pl.ds(k * 16, 16)]
                    out_v[r, pl.ds(col, 16)] = zz + (qq - zz)
                    dd = zz - qq
                    a = a + dd * dd
            return a
        acc = lax.fori_loop(0, ze_per_ch, _row, acc)

        pltpu.sync_copy(ones_v, hist_sh.at[idx_v], add=True)

        pltpu.sync_copy(out_v, quant_hbm.at[pl.ds(ze0, ze_per_ch), :])
        return acc

    acc = lax.fori_loop(0, nch, _chunk, jnp.zeros((16,), jnp.float32))

    sse_v[...] = acc
    pltpu.sync_copy(sse_v, sse_hbm.at[wid])

    plsc.subcore_barrier()

    @pl.when(s_id == 0)
    def _():
        pltpu.sync_copy(hist_sh, hist_hbm.at[c_id])


def _sc_vq(idx_flat, codebook, z_e):
    b, e = z_e.shape
    n_rows = b * (e // _D)
    cb_pad = jnp.concatenate(
        [codebook, jnp.zeros((_K, 128 - _D), jnp.float32)], axis=1)
    ones = jnp.ones((_CH, 1), jnp.float32)
    zeros = jnp.zeros((_K, 1), jnp.float32)
    kern = functools.partial(
        pl.kernel,
        mesh=plsc.VectorSubcoreMesh(core_axis_name="c", subcore_axis_name="s"),
        out_type=[
            jax.ShapeDtypeStruct((b, e), jnp.float32),
            jax.ShapeDtypeStruct((_NC, _K, 1), jnp.float32),
            jax.ShapeDtypeStruct((_NW, 16), jnp.float32),
        ],
        scratch_types=[
            pltpu.VMEM((_CH,), jnp.int32),
            pltpu.VMEM((_CH, 128), jnp.float32),
            pltpu.VMEM((_CH // 8, 512), jnp.float32),
            pltpu.VMEM((_CH // 8, 512), jnp.float32),
            pltpu.VMEM((_CH, 1), jnp.float32),
            pltpu.VMEM((16,), jnp.float32),
            pltpu.VMEM_SHARED((_K, 1), jnp.float32),
            pltpu.SemaphoreType.DMA,
        ],
    )(_sc_vq_body)
    return kern(idx_flat, cb_pad, z_e, ones, zeros)


def _finish_body(hist_ref, sse_ref, loss_ref, perp_ref, *, n_rows):
    counts = jnp.sum(hist_ref[...], axis=0, keepdims=True)
    avg = counts * (1.0 / n_rows)
    ent = jnp.sum(avg * jnp.log(avg + 1e-10))
    perp_ref[0, 0] = jnp.exp(-ent)
    loss_ref[0, 0] = jnp.sum(sse_ref[...]) / (n_rows * _D) * 0.1


def kernel(z_e, codebook):
    b, e = z_e.shape
    n_rows = b * (e // _D)
    blk_e = 128
    grid = b // blk_e

    z = z_e.reshape(-1, _D)
    x2 = jnp.sum(z * z, axis=1).reshape(b, e // _D)
    w2 = jnp.sum(codebook * codebook, axis=1)[None, :]

    idx48 = pl.pallas_call(
        functools.partial(_argmin_body, blk_e=blk_e),
        grid=(grid,),
        in_specs=[
            pl.BlockSpec((blk_e, e), lambda i: (i, 0)),
            pl.BlockSpec((_K, _D), lambda i: (0, 0)),
            pl.BlockSpec((blk_e, e // _D), lambda i: (i, 0)),
            pl.BlockSpec((1, _K), lambda i: (0, 0)),
        ],
        out_specs=pl.BlockSpec((blk_e, e // _D), lambda i: (i, 0)),
        out_shape=jax.ShapeDtypeStruct((b, e // _D), jnp.int32),
    )(z_e, codebook, x2, w2)

    idx_flat = idx48.reshape(n_rows)
    quant, hist, sse = _sc_vq(idx_flat, codebook, z_e)
    hist = hist.reshape(_NC, _K)

    loss, perp = pl.pallas_call(
        functools.partial(_finish_body, n_rows=n_rows),
        out_specs=[
            pl.BlockSpec(memory_space=pltpu.SMEM),
            pl.BlockSpec(memory_space=pltpu.SMEM),
        ],
        out_shape=[
            jax.ShapeDtypeStruct((1, 1), jnp.float32),
            jax.ShapeDtypeStruct((1, 1), jnp.float32),
        ],
    )(hist, sse)

    return (quant, loss[0, 0], idx48, perp[0, 0])


# speedup vs baseline: 1.0758x; 1.0758x over previous
import functools

import jax
import jax.numpy as jnp
from jax import lax
from jax.experimental import pallas as pl
from jax.experimental.pallas import tpu as pltpu
from jax.experimental.pallas import tpu_sc as plsc

_D = 64
_K = 1024
_BLK = 512

_NC = 2
_NS = 16
_NW = _NC * _NS
_CH = 128


def _argmin_body(z_ref, cb_ref, x2_ref, w2_ref, idx_ref, *, blk_e):
    cb = cb_ref[...]
    ze = z_ref[...]
    x2_all = x2_ref[...]
    w2 = w2_ref[...]
    lane = jax.lax.broadcasted_iota(jnp.int32, (blk_e, _K), 1)

    cols = []
    for s in range(8):
        z = jax.lax.slice(ze, (0, s * _D), (blk_e, (s + 1) * _D))
        x2 = jax.lax.slice(x2_all, (0, s), (blk_e, s + 1))

        zc = jax.lax.dot_general(z, cb, (((1,), (1,)), ((), ())),
                                 preferred_element_type=jnp.float32)
        d2 = jnp.maximum(x2 - 2.0 * zc + w2, 0.0)
        dist = jnp.sqrt(d2)

        minval = jnp.min(dist, axis=1, keepdims=True)
        cols.append(jnp.min(jnp.where(dist == minval, lane, _K), axis=1,
                            keepdims=True))
    idx_ref[...] = jnp.concatenate(cols, axis=1)


def _sc_vq_body(idx_hbm, cb_hbm, z_hbm, ones_hbm, zeros_hbm,
                quant_hbm, hist_hbm, sse_hbm,
                idx_v0, idx_v1, rows_v0, rows_v1, z_v, out_v, ones_v, sse_v,
                hist_sh, sem0, sem1):
    idx_v = (idx_v0, idx_v1)
    rows_v = (rows_v0, rows_v1)
    sem = (sem0, sem1)
    c_id = lax.axis_index("c")
    s_id = lax.axis_index("s")
    wid = s_id * _NC + c_id
    n_rows = z_hbm.shape[0] * (z_hbm.shape[1] // _D)
    rpw = n_rows // _NW
    nch = rpw // _CH
    base = wid * rpw

    pltpu.sync_copy(ones_hbm, ones_v)

    @pl.when(s_id == 0)
    def _():
        pltpu.sync_copy(zeros_hbm, hist_sh)
    plsc.subcore_barrier()

    ze_per_ch = _CH // 8

    def _start_gather(c, buf):
        row0 = pl.multiple_of(base + c * _CH, _CH)
        pltpu.sync_copy(idx_hbm.at[pl.ds(row0, _CH)], idx_v[buf])
        return pltpu.async_copy(cb_hbm.at[idx_v[buf]], rows_v[buf], sem[buf])

    acc = jnp.zeros((16,), jnp.float32)
    handle = _start_gather(0, 0)
    for c in range(nch):
        cur, nxt = c % 2, (c + 1) % 2
        if c + 1 < nch:
            next_handle = _start_gather(c + 1, nxt)
        ze0 = pl.multiple_of(base // 8 + c * ze_per_ch, ze_per_ch)
        pltpu.sync_copy(z_hbm.at[pl.ds(ze0, ze_per_ch), :], z_v)
        handle.wait()

        def _row(r, a, cur=cur):
            for s in range(8):
                for k in range(_D // 16):
                    col = s * _D + k * 16
                    zz = z_v[r, pl.ds(col, 16)]
                    qq = rows_v[cur][r * 8 + s, pl.ds(k * 16, 16)]
                    out_v[r, pl.ds(col, 16)] = zz + (qq - zz)
                    dd = zz - qq
                    a = a + dd * dd
            return a
        acc = lax.fori_loop(0, ze_per_ch, _row, acc)

        pltpu.sync_copy(ones_v, hist_sh.at[idx_v[cur]], add=True)

        pltpu.sync_copy(out_v, quant_hbm.at[pl.ds(ze0, ze_per_ch), :])
        if c + 1 < nch:
            handle = next_handle

    sse_v[...] = acc
    pltpu.sync_copy(sse_v, sse_hbm.at[wid])

    plsc.subcore_barrier()

    @pl.when(s_id == 0)
    def _():
        pltpu.sync_copy(hist_sh, hist_hbm.at[c_id])


def _sc_vq(idx_flat, codebook, z_e):
    b, e = z_e.shape
    n_rows = b * (e // _D)
    cb_pad = jnp.concatenate(
        [codebook, jnp.zeros((_K, 128 - _D), jnp.float32)], axis=1)
    ones = jnp.ones((_CH, 1), jnp.float32)
    zeros = jnp.zeros((_K, 1), jnp.float32)
    kern = functools.partial(
        pl.kernel,
        mesh=plsc.VectorSubcoreMesh(core_axis_name="c", subcore_axis_name="s"),
        out_type=[
            jax.ShapeDtypeStruct((b, e), jnp.float32),
            jax.ShapeDtypeStruct((_NC, _K, 1), jnp.float32),
            jax.ShapeDtypeStruct((_NW, 16), jnp.float32),
        ],
        scratch_types=[
            pltpu.VMEM((_CH,), jnp.int32),
            pltpu.VMEM((_CH,), jnp.int32),
            pltpu.VMEM((_CH, 128), jnp.float32),
            pltpu.VMEM((_CH, 128), jnp.float32),
            pltpu.VMEM((_CH // 8, 512), jnp.float32),
            pltpu.VMEM((_CH // 8, 512), jnp.float32),
            pltpu.VMEM((_CH, 1), jnp.float32),
            pltpu.VMEM((16,), jnp.float32),
            pltpu.VMEM_SHARED((_K, 1), jnp.float32),
            pltpu.SemaphoreType.DMA,
            pltpu.SemaphoreType.DMA,
        ],
    )(_sc_vq_body)
    return kern(idx_flat, cb_pad, z_e, ones, zeros)


def _finish_body(hist_ref, sse_ref, loss_ref, perp_ref, *, n_rows):
    counts = jnp.sum(hist_ref[...], axis=0, keepdims=True)
    avg = counts * (1.0 / n_rows)
    ent = jnp.sum(avg * jnp.log(avg + 1e-10))
    perp_ref[0, 0] = jnp.exp(-ent)
    loss_ref[0, 0] = jnp.sum(sse_ref[...]) / (n_rows * _D) * 0.1


def kernel(z_e, codebook):
    b, e = z_e.shape
    n_rows = b * (e // _D)
    blk_e = 128
    grid = b // blk_e

    ze3 = z_e.reshape(b, e // _D, _D)
    x2 = jnp.sum(ze3 * ze3, axis=2)
    w2 = jnp.sum(codebook * codebook, axis=1)[None, :]

    idx48 = pl.pallas_call(
        functools.partial(_argmin_body, blk_e=blk_e),
        grid=(grid,),
        in_specs=[
            pl.BlockSpec((blk_e, e), lambda i: (i, 0)),
            pl.BlockSpec((_K, _D), lambda i: (0, 0)),
            pl.BlockSpec((blk_e, e // _D), lambda i: (i, 0)),
            pl.BlockSpec((1, _K), lambda i: (0, 0)),
        ],
        out_specs=pl.BlockSpec((blk_e, e // _D), lambda i: (i, 0)),
        out_shape=jax.ShapeDtypeStruct((b, e // _D), jnp.int32),
    )(z_e, codebook, x2, w2)

    idx_flat = idx48.reshape(n_rows)
    quant, hist, sse = _sc_vq(idx_flat, codebook, z_e)
    hist = hist.reshape(_NC, _K)

    loss, perp = pl.pallas_call(
        functools.partial(_finish_body, n_rows=n_rows),
        out_specs=[
            pl.BlockSpec(memory_space=pltpu.SMEM),
            pl.BlockSpec(memory_space=pltpu.SMEM),
        ],
        out_shape=[
            jax.ShapeDtypeStruct((1, 1), jnp.float32),
            jax.ShapeDtypeStruct((1, 1), jnp.float32),
        ],
    )(hist, sse)

    return (quant, loss[0, 0], idx48, perp[0, 0])


# speedup vs baseline: 1.0984x; 1.0210x over previous
import functools

import jax
import jax.numpy as jnp
from jax import lax
from jax.experimental import pallas as pl
from jax.experimental.pallas import tpu as pltpu
from jax.experimental.pallas import tpu_sc as plsc

_D = 64
_K = 1024
_BLK = 512

_NC = 2
_NS = 16
_NW = _NC * _NS
_CH = 128


def _argmin_body(z_ref, cb_ref, x2_ref, w2_ref, idx_ref, counts_ref,
                 counts_acc, *, blk_e):
    i = pl.program_id(0)
    g = pl.num_programs(0)
    cb = cb_ref[...]
    ze = z_ref[...]
    x2_all = x2_ref[...]
    w2 = w2_ref[...]
    lane = jax.lax.broadcasted_iota(jnp.int32, (blk_e, _K), 1)

    @pl.when(i == 0)
    def _():
        counts_acc[...] = jnp.zeros_like(counts_acc)

    cols = []
    bc = None
    for s in range(8):
        z = jax.lax.slice(ze, (0, s * _D), (blk_e, (s + 1) * _D))
        x2 = jax.lax.slice(x2_all, (0, s), (blk_e, s + 1))

        zc = jax.lax.dot_general(z, cb, (((1,), (1,)), ((), ())),
                                 preferred_element_type=jnp.float32)
        d2 = jnp.maximum(x2 - 2.0 * zc + w2, 0.0)
        dist = jnp.sqrt(d2)

        minval = jnp.min(dist, axis=1, keepdims=True)
        idx = jnp.min(jnp.where(dist == minval, lane, _K), axis=1,
                      keepdims=True)
        cols.append(idx)
        onehot = (lane == idx).astype(jnp.float32)
        s_cnt = jnp.sum(onehot, axis=0, keepdims=True)
        bc = s_cnt if bc is None else bc + s_cnt
    idx_ref[...] = jnp.concatenate(cols, axis=1)
    counts_acc[...] += bc

    @pl.when(i == g - 1)
    def _():
        counts_ref[...] = counts_acc[...]


def _sc_vq_body(idx_hbm, cb_hbm, z_hbm,
                quant_hbm, sse_hbm,
                idx_all, rows_v0, rows_v1, z_v, out_v, sse_v,
                sem0, sem1):
    rows_v = (rows_v0, rows_v1)
    sem = (sem0, sem1)
    c_id = lax.axis_index("c")
    s_id = lax.axis_index("s")
    wid = s_id * _NC + c_id
    n_rows = idx_hbm.shape[0] * idx_hbm.shape[1]
    rpw = n_rows // _NW
    nch = rpw // _CH

    irow0 = pl.multiple_of(wid * nch, nch)
    pltpu.sync_copy(idx_hbm.at[pl.ds(irow0, nch), :], idx_all)

    base = wid * rpw
    ze_per_ch = _CH // 8

    def _start_gather(c, buf):
        return pltpu.async_copy(cb_hbm.at[idx_all.at[c]], rows_v[buf],
                                sem[buf])

    acc = jnp.zeros((16,), jnp.float32)
    handle = _start_gather(0, 0)
    for c in range(nch):
        cur, nxt = c % 2, (c + 1) % 2
        if c + 1 < nch:
            next_handle = _start_gather(c + 1, nxt)
        ze0 = pl.multiple_of(base // 8 + c * ze_per_ch, ze_per_ch)
        pltpu.sync_copy(z_hbm.at[pl.ds(ze0, ze_per_ch), :], z_v)
        handle.wait()

        def _row(r, a, cur=cur):
            for s in range(8):
                for k in range(_D // 16):
                    col = s * _D + k * 16
                    zz = z_v[r, pl.ds(col, 16)]
                    qq = rows_v[cur][r * 8 + s, pl.ds(k * 16, 16)]
                    out_v[r, pl.ds(col, 16)] = zz + (qq - zz)
                    dd = zz - qq
                    a = a + dd * dd
            return a
        acc = lax.fori_loop(0, ze_per_ch, _row, acc)

        pltpu.sync_copy(out_v, quant_hbm.at[pl.ds(ze0, ze_per_ch), :])
        if c + 1 < nch:
            handle = next_handle

    sse_v[...] = acc
    pltpu.sync_copy(sse_v, sse_hbm.at[wid])


def _sc_vq(idx_flat, codebook, z_e):
    b, e = z_e.shape
    n_rows = b * (e // _D)
    cb_pad = jnp.concatenate(
        [codebook, jnp.zeros((_K, 128 - _D), jnp.float32)], axis=1)
    kern = functools.partial(
        pl.kernel,
        mesh=plsc.VectorSubcoreMesh(core_axis_name="c", subcore_axis_name="s"),
        out_type=[
            jax.ShapeDtypeStruct((b, e), jnp.float32),
            jax.ShapeDtypeStruct((_NW, 16), jnp.float32),
        ],
        scratch_types=[
            pltpu.VMEM((n_rows // _NW // _CH, _CH), jnp.int32),
            pltpu.VMEM((_CH, 128), jnp.float32),
            pltpu.VMEM((_CH, 128), jnp.float32),
            pltpu.VMEM((_CH // 8, 512), jnp.float32),
            pltpu.VMEM((_CH // 8, 512), jnp.float32),
            pltpu.VMEM((16,), jnp.float32),
            pltpu.SemaphoreType.DMA,
            pltpu.SemaphoreType.DMA,
        ],
    )(_sc_vq_body)
    return kern(idx_flat, cb_pad, z_e)


def _finish_body(counts_ref, sse_ref, loss_ref, perp_ref, *, n_rows):
    avg = counts_ref[...] * (1.0 / n_rows)
    ent = jnp.sum(avg * jnp.log(avg + 1e-10))
    perp_ref[0, 0] = jnp.exp(-ent)
    loss_ref[0, 0] = jnp.sum(sse_ref[...]) / (n_rows * _D) * 0.1


def kernel(z_e, codebook):
    b, e = z_e.shape
    n_rows = b * (e // _D)
    blk_e = 128
    grid = b // blk_e

    ze3 = z_e.reshape(b, e // _D, _D)
    x2 = jnp.sum(ze3 * ze3, axis=2)
    w2 = jnp.sum(codebook * codebook, axis=1)[None, :]

    idx48, counts = pl.pallas_call(
        functools.partial(_argmin_body, blk_e=blk_e),
        grid=(grid,),
        in_specs=[
            pl.BlockSpec((blk_e, e), lambda i: (i, 0)),
            pl.BlockSpec((_K, _D), lambda i: (0, 0)),
            pl.BlockSpec((blk_e, e // _D), lambda i: (i, 0)),
            pl.BlockSpec((1, _K), lambda i: (0, 0)),
        ],
        out_specs=[
            pl.BlockSpec((blk_e, e // _D), lambda i: (i, 0)),
            pl.BlockSpec((1, _K), lambda i: (0, 0)),
        ],
        out_shape=[
            jax.ShapeDtypeStruct((b, e // _D), jnp.int32),
            jax.ShapeDtypeStruct((1, _K), jnp.float32),
        ],
        scratch_shapes=[pltpu.VMEM((1, _K), jnp.float32)],
    )(z_e, codebook, x2, w2)

    idx_flat = idx48.reshape(n_rows // _CH, _CH)
    quant, sse = _sc_vq(idx_flat, codebook, z_e)

    loss, perp = pl.pallas_call(
        functools.partial(_finish_body, n_rows=n_rows),
        out_specs=[
            pl.BlockSpec(memory_space=pltpu.SMEM),
            pl.BlockSpec(memory_space=pltpu.SMEM),
        ],
        out_shape=[
            jax.ShapeDtypeStruct((1, 1), jnp.float32),
            jax.ShapeDtypeStruct((1, 1), jnp.float32),
        ],
    )(counts, sse)

    return (quant, loss[0, 0], idx48, perp[0, 0])


# speedup vs baseline: 1.1480x; 1.0452x over previous
import functools

import jax
import jax.numpy as jnp
from jax import lax
from jax.experimental import pallas as pl
from jax.experimental.pallas import tpu as pltpu
from jax.experimental.pallas import tpu_sc as plsc

_D = 64
_K = 1024
_BLK = 512

_NC = 2
_NS = 16
_NW = _NC * _NS
_CH = 128


def _argmin_body(z_ref, cb_ref, x2_ref, w2_ref, idx_ref, counts_ref,
                 counts_acc, *, blk_e):
    i = pl.program_id(0)
    g = pl.num_programs(0)
    cb = cb_ref[...]
    ze = z_ref[...]
    x2_all = x2_ref[...]
    w2 = w2_ref[...]
    lane = jax.lax.broadcasted_iota(jnp.int32, (blk_e, _K), 1)

    @pl.when(i == 0)
    def _():
        counts_acc[...] = jnp.zeros_like(counts_acc)

    cols = []
    bc = None
    for s in range(8):
        z = jax.lax.slice(ze, (0, s * _D), (blk_e, (s + 1) * _D))
        x2 = jax.lax.slice(x2_all, (0, s), (blk_e, s + 1))

        zc = jax.lax.dot_general(z, cb, (((1,), (1,)), ((), ())),
                                 preferred_element_type=jnp.float32)
        d2 = jnp.maximum(x2 - 2.0 * zc + w2, 0.0)
        dist = jnp.sqrt(d2)

        minval = jnp.min(dist, axis=1, keepdims=True)
        idx = jnp.min(jnp.where(dist == minval, lane, _K), axis=1,
                      keepdims=True)
        cols.append(idx)
        onehot = (lane == idx).astype(jnp.float32)
        s_cnt = jnp.sum(onehot, axis=0, keepdims=True)
        bc = s_cnt if bc is None else bc + s_cnt
    idx_ref[...] = jnp.concatenate(cols, axis=1)
    counts_acc[...] += bc

    @pl.when(i == g - 1)
    def _():
        counts_ref[...] = counts_acc[...]


def _sc_vq_body(idx_hbm, cb_hbm, z_hbm,
                quant_hbm, sse_hbm,
                idx_all, rows_v0, rows_v1, z_v, out_v, sse_v,
                sem0, sem1):
    rows_v = (rows_v0, rows_v1)
    sem = (sem0, sem1)
    c_id = lax.axis_index("c")
    s_id = lax.axis_index("s")
    wid = s_id * _NC + c_id
    n_rows = idx_hbm.shape[0] * idx_hbm.shape[1]
    rpw = n_rows // _NW
    nch = rpw // _CH

    irow0 = pl.multiple_of(wid * nch, nch)
    pltpu.sync_copy(idx_hbm.at[pl.ds(irow0, nch), :], idx_all)

    base = wid * rpw
    ze_per_ch = _CH // 8

    def _start_gather(c, buf):
        return pltpu.async_copy(cb_hbm.at[idx_all.at[c]], rows_v[buf],
                                sem[buf])

    acc = jnp.zeros((16,), jnp.float32)
    handle = _start_gather(0, 0)
    for c in range(nch):
        cur, nxt = c % 2, (c + 1) % 2
        if c + 1 < nch:
            next_handle = _start_gather(c + 1, nxt)
        ze0 = pl.multiple_of(base // 8 + c * ze_per_ch, ze_per_ch)
        pltpu.sync_copy(z_hbm.at[pl.ds(ze0, ze_per_ch), :], z_v)
        handle.wait()

        def _row(r, a, cur=cur):
            for s in range(8):
                for k in range(_D // 16):
                    col = s * _D + k * 16
                    zz = z_v[r, pl.ds(col, 16)]
                    qq = rows_v[cur][r * 8 + s, pl.ds(k * 16, 16)]
                    out_v[r, pl.ds(col, 16)] = zz + (qq - zz)
                    dd = zz - qq
                    a = a + dd * dd
            return a
        acc = lax.fori_loop(0, ze_per_ch, _row, acc)

        pltpu.sync_copy(out_v, quant_hbm.at[pl.ds(ze0, ze_per_ch), :])
        if c + 1 < nch:
            handle = next_handle

    sse_v[...] = acc
    pltpu.sync_copy(sse_v, sse_hbm.at[wid])


def _sc_vq(idx_flat, codebook, z_e):
    b, e = z_e.shape
    n_rows = b * (e // _D)
    cb_pad = jnp.concatenate(
        [codebook, jnp.zeros((_K, 128 - _D), jnp.float32)], axis=1)
    kern = functools.partial(
        pl.kernel,
        mesh=plsc.VectorSubcoreMesh(core_axis_name="c", subcore_axis_name="s"),
        out_type=[
            jax.ShapeDtypeStruct((b, e), jnp.float32),
            jax.ShapeDtypeStruct((_NW, 16), jnp.float32),
        ],
        scratch_types=[
            pltpu.VMEM((n_rows // _NW // _CH, _CH), jnp.int32),
            pltpu.VMEM((_CH, 128), jnp.float32),
            pltpu.VMEM((_CH, 128), jnp.float32),
            pltpu.VMEM((_CH // 8, 512), jnp.float32),
            pltpu.VMEM((_CH // 8, 512), jnp.float32),
            pltpu.VMEM((16,), jnp.float32),
            pltpu.SemaphoreType.DMA,
            pltpu.SemaphoreType.DMA,
        ],
    )(_sc_vq_body)
    return kern(idx_flat, cb_pad, z_e)


def _finish_body(counts_ref, sse_ref, loss_ref, perp_ref, *, n_rows):
    avg = counts_ref[...] * (1.0 / n_rows)
    ent = jnp.sum(avg * jnp.log(avg + 1e-10))
    perp_ref[0, 0] = jnp.exp(-ent)
    loss_ref[0, 0] = jnp.sum(sse_ref[...]) / (n_rows * _D) * 0.1


def kernel(z_e, codebook):
    b, e = z_e.shape
    n_rows = b * (e // _D)
    blk_e = 256
    grid = b // blk_e

    ze3 = z_e.reshape(b, e // _D, _D)
    x2 = jnp.sum(ze3 * ze3, axis=2)
    w2 = jnp.sum(codebook * codebook, axis=1)[None, :]

    idx48, counts = pl.pallas_call(
        functools.partial(_argmin_body, blk_e=blk_e),
        grid=(grid,),
        in_specs=[
            pl.BlockSpec((blk_e, e), lambda i: (i, 0)),
            pl.BlockSpec((_K, _D), lambda i: (0, 0)),
            pl.BlockSpec((blk_e, e // _D), lambda i: (i, 0)),
            pl.BlockSpec((1, _K), lambda i: (0, 0)),
        ],
        out_specs=[
            pl.BlockSpec((blk_e, e // _D), lambda i: (i, 0)),
            pl.BlockSpec((1, _K), lambda i: (0, 0)),
        ],
        out_shape=[
            jax.ShapeDtypeStruct((b, e // _D), jnp.int32),
            jax.ShapeDtypeStruct((1, _K), jnp.float32),
        ],
        scratch_shapes=[pltpu.VMEM((1, _K), jnp.float32)],
    )(z_e, codebook, x2, w2)

    idx_flat = idx48.reshape(n_rows // _CH, _CH)
    quant, sse = _sc_vq(idx_flat, codebook, z_e)

    loss, perp = pl.pallas_call(
        functools.partial(_finish_body, n_rows=n_rows),
        out_specs=[
            pl.BlockSpec(memory_space=pltpu.SMEM),
            pl.BlockSpec(memory_space=pltpu.SMEM),
        ],
        out_shape=[
            jax.ShapeDtypeStruct((1, 1), jnp.float32),
            jax.ShapeDtypeStruct((1, 1), jnp.float32),
        ],
    )(counts, sse)

    return (quant, loss[0, 0], idx48, perp[0, 0])


# speedup vs baseline: 1.1558x; 1.0067x over previous
import functools

import jax
import jax.numpy as jnp
from jax import lax
from jax.experimental import pallas as pl
from jax.experimental.pallas import tpu as pltpu
from jax.experimental.pallas import tpu_sc as plsc

_D = 64
_K = 1024
_BLK = 512

_NC = 2
_NS = 16
_NW = _NC * _NS
_CH = 128


def _argmin_body(z_ref, cb_ref, x2_ref, w2_ref, idx_ref, counts_ref,
                 counts_acc, *, blk_e):
    i = pl.program_id(0)
    g = pl.num_programs(0)
    cb = cb_ref[...]
    ze = z_ref[...]
    x2_all = x2_ref[...]
    w2 = w2_ref[...]
    lane = jax.lax.broadcasted_iota(jnp.int32, (blk_e, _K), 1)

    @pl.when(i == 0)
    def _():
        counts_acc[...] = jnp.zeros_like(counts_acc)

    cols = []
    bc = None
    for s in range(8):
        z = jax.lax.slice(ze, (0, s * _D), (blk_e, (s + 1) * _D))
        x2 = jax.lax.slice(x2_all, (0, s), (blk_e, s + 1))

        zc = jax.lax.dot_general(z, cb, (((1,), (1,)), ((), ())),
                                 preferred_element_type=jnp.float32)
        d2 = jnp.maximum(x2 - 2.0 * zc + w2, 0.0)
        dist = jnp.sqrt(d2)

        minval = jnp.min(dist, axis=1, keepdims=True)
        idx = jnp.min(jnp.where(dist == minval, lane, _K), axis=1,
                      keepdims=True)
        cols.append(idx)
        onehot = (lane == idx).astype(jnp.float32)
        s_cnt = jnp.sum(onehot, axis=0, keepdims=True)
        bc = s_cnt if bc is None else bc + s_cnt
    idx_ref[...] = jnp.concatenate(cols, axis=1)
    counts_acc[...] += bc

    @pl.when(i == g - 1)
    def _():
        counts_ref[...] = counts_acc[...]


def _sc_vq_body(idx_hbm, cb_hbm, z_hbm,
                quant_hbm, sse_hbm,
                idx_all, rows_v0, rows_v1, z_v, out_v, sse_v,
                sem0, sem1):
    rows_v = (rows_v0, rows_v1)
    sem = (sem0, sem1)
    c_id = lax.axis_index("c")
    s_id = lax.axis_index("s")
    wid = s_id * _NC + c_id
    n_rows = idx_hbm.shape[0] * idx_hbm.shape[1]
    rpw = n_rows // _NW
    nch = rpw // _CH

    irow0 = pl.multiple_of(wid * nch, nch)
    pltpu.sync_copy(idx_hbm.at[pl.ds(irow0, nch), :], idx_all)

    base = wid * rpw
    ze_per_ch = _CH // 8

    def _start_gather(c, buf):
        return pltpu.async_copy(cb_hbm.at[idx_all.at[c]], rows_v[buf],
                                sem[buf])

    acc = jnp.zeros((16,), jnp.float32)
    handle = _start_gather(0, 0)
    for c in range(nch):
        cur, nxt = c % 2, (c + 1) % 2
        if c + 1 < nch:
            next_handle = _start_gather(c + 1, nxt)
        ze0 = pl.multiple_of(base // 8 + c * ze_per_ch, ze_per_ch)
        pltpu.sync_copy(z_hbm.at[pl.ds(ze0, ze_per_ch), :], z_v)
        handle.wait()

        def _row(r, a, cur=cur):
            for s in range(8):
                for k in range(_D // 16):
                    col = s * _D + k * 16
                    zz = z_v[r, pl.ds(col, 16)]
                    qq = rows_v[cur][r * 8 + s, pl.ds(k * 16, 16)]
                    out_v[r, pl.ds(col, 16)] = zz + (qq - zz)
                    dd = zz - qq
                    a = a + dd * dd
            return a
        acc = lax.fori_loop(0, ze_per_ch, _row, acc)

        pltpu.sync_copy(out_v, quant_hbm.at[pl.ds(ze0, ze_per_ch), :])
        if c + 1 < nch:
            handle = next_handle

    sse_v[...] = acc
    pltpu.sync_copy(sse_v, sse_hbm.at[wid])


def _sc_vq(idx_flat, codebook, z_e):
    b, e = z_e.shape
    n_rows = b * (e // _D)
    cb_pad = jnp.concatenate(
        [codebook, jnp.zeros((_K, 128 - _D), jnp.float32)], axis=1)
    kern = functools.partial(
        pl.kernel,
        mesh=plsc.VectorSubcoreMesh(core_axis_name="c", subcore_axis_name="s"),
        out_type=[
            jax.ShapeDtypeStruct((b, e), jnp.float32),
            jax.ShapeDtypeStruct((_NW, 16), jnp.float32),
        ],
        scratch_types=[
            pltpu.VMEM((n_rows // _NW // _CH, _CH), jnp.int32),
            pltpu.VMEM((_CH, 128), jnp.float32),
            pltpu.VMEM((_CH, 128), jnp.float32),
            pltpu.VMEM((_CH // 8, 512), jnp.float32),
            pltpu.VMEM((_CH // 8, 512), jnp.float32),
            pltpu.VMEM((16,), jnp.float32),
            pltpu.SemaphoreType.DMA,
            pltpu.SemaphoreType.DMA,
        ],
    )(_sc_vq_body)
    return kern(idx_flat, cb_pad, z_e)


def _finish_body(counts_ref, sse_ref, loss_ref, perp_ref, *, n_rows):
    avg = counts_ref[...] * (1.0 / n_rows)
    ent = jnp.sum(avg * jnp.log(avg + 1e-10))
    perp_ref[0, 0] = jnp.exp(-ent)
    loss_ref[0, 0] = jnp.sum(sse_ref[...]) / (n_rows * _D) * 0.1


def kernel(z_e, codebook):
    b, e = z_e.shape
    n_rows = b * (e // _D)
    blk_e = 512
    grid = b // blk_e

    ze3 = z_e.reshape(b, e // _D, _D)
    x2 = jnp.sum(ze3 * ze3, axis=2)
    w2 = jnp.sum(codebook * codebook, axis=1)[None, :]

    idx48, counts = pl.pallas_call(
        functools.partial(_argmin_body, blk_e=blk_e),
        grid=(grid,),
        in_specs=[
            pl.BlockSpec((blk_e, e), lambda i: (i, 0)),
            pl.BlockSpec((_K, _D), lambda i: (0, 0)),
            pl.BlockSpec((blk_e, e // _D), lambda i: (i, 0)),
            pl.BlockSpec((1, _K), lambda i: (0, 0)),
        ],
        out_specs=[
            pl.BlockSpec((blk_e, e // _D), lambda i: (i, 0)),
            pl.BlockSpec((1, _K), lambda i: (0, 0)),
        ],
        out_shape=[
            jax.ShapeDtypeStruct((b, e // _D), jnp.int32),
            jax.ShapeDtypeStruct((1, _K), jnp.float32),
        ],
        scratch_shapes=[pltpu.VMEM((1, _K), jnp.float32)],
    )(z_e, codebook, x2, w2)

    idx_flat = idx48.reshape(n_rows // _CH, _CH)
    quant, sse = _sc_vq(idx_flat, codebook, z_e)

    loss, perp = pl.pallas_call(
        functools.partial(_finish_body, n_rows=n_rows),
        out_specs=[
            pl.BlockSpec(memory_space=pltpu.SMEM),
            pl.BlockSpec(memory_space=pltpu.SMEM),
        ],
        out_shape=[
            jax.ShapeDtypeStruct((1, 1), jnp.float32),
            jax.ShapeDtypeStruct((1, 1), jnp.float32),
        ],
    )(counts, sse)

    return (quant, loss[0, 0], idx48, perp[0, 0])
